# baseline (speedup 1.0000x reference)
import functools

import jax
import jax.numpy as jnp
from jax import lax
from jax.experimental import pallas as pl
from jax.experimental.pallas import tpu as pltpu
from jax.experimental.pallas import tpu_sc as plsc

NUM_EMB = 8192
DIM = 256
CCOST = 0.25
TM = 512


LANES = 128


def _dist_argmin_body(zt_ref, w_ref, idx_ref, minsum_ref, wsq_ref):
    b = pl.program_id(0)
    t = pl.program_id(1)
    first = (b == 0) & (t == 0)

    @pl.when(first)
    def _():
        w3 = w_ref[...].reshape(NUM_EMB // LANES, LANES, DIM)
        wsq_ref[...] = jnp.sum(w3 * w3, axis=2)

    z = jnp.transpose(zt_ref[0], (1, 0))
    zsq = jnp.sum(z * z, axis=1, keepdims=True)
    mm = lax.dot_general(z, w_ref[...], (((1,), (1,)), ((), ())),
                         preferred_element_type=jnp.float32)
    wsq = wsq_ref[...]

    best = (zsq - 2.0 * mm[:, :LANES]) + wsq[0][None, :]
    bestj = jnp.zeros((TM, LANES), jnp.int32)
    for j in range(1, NUM_EMB // LANES):
        s = (zsq - 2.0 * mm[:, j * LANES:(j + 1) * LANES]) + wsq[j][None, :]
        m = s < best
        best = jnp.where(m, s, best)
        bestj = jnp.where(m, j, bestj)
    lane = lax.broadcasted_iota(jnp.int32, (TM, LANES), 1)
    absidx = bestj * LANES + lane
    rowmin = jnp.min(best, axis=1, keepdims=True)
    idx = jnp.min(jnp.where(best == rowmin, absidx, NUM_EMB), axis=1)
    idx_ref[...] = idx
    s = jnp.sum(rowmin)

    @pl.when(first)
    def _():
        minsum_ref[0, 0] = s

    @pl.when(jnp.logical_not(first))
    def _():
        minsum_ref[0, 0] = minsum_ref[0, 0] + s


def _dist_argmin(z3, w):
    nb = z3.shape[0]
    nt = z3.shape[2] // TM
    return pl.pallas_call(
        _dist_argmin_body,
        grid=(nb, nt),
        in_specs=[
            pl.BlockSpec((1, DIM, TM), lambda b, t: (b, 0, t)),
            pl.BlockSpec((NUM_EMB, DIM), lambda b, t: (0, 0)),
        ],
        out_specs=[
            pl.BlockSpec((TM,), lambda b, t: (b * nt + t,)),
            pl.BlockSpec((1, 1), lambda b, t: (0, 0), memory_space=pltpu.SMEM),
        ],
        out_shape=[
            jax.ShapeDtypeStruct((nb * nt * TM,), jnp.int32),
            jax.ShapeDtypeStruct((1, 1), jnp.float32),
        ],
        scratch_shapes=[pltpu.VMEM((NUM_EMB // LANES, LANES), jnp.float32)],
    )(z3, w)


def _ste_body(zq_ref, z3_ref, out_ref):
    zq_t = jnp.transpose(zq_ref[...], (1, 0))
    zb = z3_ref[0]
    out_ref[0] = zb + (zq_t - zb)


def _ste_transpose(zq_flat, z3):
    nb = z3.shape[0]
    nt = z3.shape[2] // TM
    return pl.pallas_call(
        _ste_body,
        grid=(nb, nt),
        in_specs=[
            pl.BlockSpec((TM, DIM), lambda b, t: (b * nt + t, 0)),
            pl.BlockSpec((1, DIM, TM), lambda b, t: (b, 0, t)),
        ],
        out_specs=pl.BlockSpec((1, DIM, TM), lambda b, t: (b, 0, t)),
        out_shape=jax.ShapeDtypeStruct(z3.shape, jnp.float32),
    )(zq_flat, z3)


_NC = 2
_NS = 16
_NW = _NC * _NS
_BPW = 8192 // _NW


@functools.cache
def _sc_gather_kernel():
    @functools.partial(
        pl.kernel,
        mesh=plsc.VectorSubcoreMesh(core_axis_name="c", subcore_axis_name="s"),
        out_type=jax.ShapeDtypeStruct((8192, DIM), jnp.float32),
        scratch_types=[
            pltpu.VMEM((_BPW,), jnp.int32),
            pltpu.VMEM((_BPW, DIM), jnp.float32),
            pltpu.SemaphoreType.DMA,
        ],
    )
    def _sc_gather(table_hbm, idx_hbm, out_hbm, idx_v, rows_v, sem):
        wid = lax.axis_index("s") * _NC + lax.axis_index("c")
        base = wid * _BPW
        pltpu.sync_copy(idx_hbm.at[pl.ds(base, _BPW)], idx_v)
        pltpu.async_copy(table_hbm.at[idx_v], rows_v, sem).wait()
        pltpu.sync_copy(rows_v, out_hbm.at[pl.ds(base, _BPW)])

    return _sc_gather


def kernel(z, W):
    B, C, D, H, Wd = z.shape
    z3 = z.reshape(B, C, D * H * Wd)
    idx, minsum = _dist_argmin(z3, W)
    z_q_flat = _sc_gather_kernel()(W, idx)
    z_q_out = _ste_transpose(z_q_flat, z3).reshape(z.shape)
    loss = (1.0 + CCOST) * (minsum[0, 0] / z.size)
    return (z_q_out, loss, idx.reshape(B, D, H, Wd))


# speedup vs baseline: 1.2058x; 1.2058x over previous
import functools

import jax
import jax.numpy as jnp
from jax import lax
from jax.experimental import pallas as pl
from jax.experimental.pallas import tpu as pltpu
from jax.experimental.pallas import tpu_sc as plsc

NUM_EMB = 8192
DIM = 256
CCOST = 0.25
TM = 512
RC = 64

LANES = 128


def _dist_argmin_body(zt_ref, w_ref, idx_ref, minsum_ref, wsq_ref):
    b = pl.program_id(0)
    t = pl.program_id(1)
    first = (b == 0) & (t == 0)

    @pl.when(first)
    def _():
        w3 = w_ref[...].reshape(NUM_EMB // LANES, LANES, DIM)
        wsq_ref[...] = jnp.sum(w3 * w3, axis=2)

    z = jnp.transpose(zt_ref[0], (1, 0))
    zsq = jnp.sum(z * z, axis=1, keepdims=True)
    mm2 = lax.dot_general(z + z, w_ref[...], (((1,), (1,)), ((), ())),
                          preferred_element_type=jnp.float32)
    wsq = wsq_ref[...]

    lane = lax.broadcasted_iota(jnp.int32, (RC, LANES), 1)
    idx_parts, s = [], jnp.float32(0.0)
    for r in range(TM // RC):
        rows = slice(r * RC, (r + 1) * RC)
        zsq_r = zsq[rows]
        best = (zsq_r - mm2[rows, :LANES]) + wsq[0][None, :]
        bestj = jnp.zeros((RC, LANES), jnp.int32)
        for j in range(1, NUM_EMB // LANES):
            sj = (zsq_r - mm2[rows, j * LANES:(j + 1) * LANES]) + wsq[j][None, :]
            m = sj < best
            best = jnp.where(m, sj, best)
            bestj = jnp.where(m, j, bestj)
        absidx = bestj * LANES + lane
        rowmin = jnp.min(best, axis=1, keepdims=True)
        idx_parts.append(
            jnp.min(jnp.where(best == rowmin, absidx, NUM_EMB), axis=1))
        s = s + jnp.sum(rowmin)
    idx_ref[...] = jnp.concatenate(idx_parts)

    @pl.when(first)
    def _():
        minsum_ref[0, 0] = s

    @pl.when(jnp.logical_not(first))
    def _():
        minsum_ref[0, 0] = minsum_ref[0, 0] + s


def _dist_argmin(z3, w):
    nb = z3.shape[0]
    nt = z3.shape[2] // TM
    return pl.pallas_call(
        _dist_argmin_body,
        grid=(nb, nt),
        in_specs=[
            pl.BlockSpec((1, DIM, TM), lambda b, t: (b, 0, t)),
            pl.BlockSpec((NUM_EMB, DIM), lambda b, t: (0, 0)),
        ],
        out_specs=[
            pl.BlockSpec((TM,), lambda b, t: (b * nt + t,)),
            pl.BlockSpec((1, 1), lambda b, t: (0, 0), memory_space=pltpu.SMEM),
        ],
        out_shape=[
            jax.ShapeDtypeStruct((nb * nt * TM,), jnp.int32),
            jax.ShapeDtypeStruct((1, 1), jnp.float32),
        ],
        scratch_shapes=[pltpu.VMEM((NUM_EMB // LANES, LANES), jnp.float32)],
    )(z3, w)


_NC = 2
_NS = 16
_NW = _NC * _NS
_BPW = 8192 // _NW


@functools.cache
def _sc_gather_kernel():
    @functools.partial(
        pl.kernel,
        mesh=plsc.VectorSubcoreMesh(core_axis_name="c", subcore_axis_name="s"),
        out_type=jax.ShapeDtypeStruct((8192, DIM), jnp.float32),
        scratch_types=[
            pltpu.VMEM((_BPW,), jnp.int32),
            pltpu.VMEM((_BPW, DIM), jnp.float32),
            pltpu.SemaphoreType.DMA,
        ],
    )
    def _sc_gather(table_hbm, idx_hbm, out_hbm, idx_v, rows_v, sem):
        wid = lax.axis_index("s") * _NC + lax.axis_index("c")
        base = wid * _BPW
        pltpu.sync_copy(idx_hbm.at[pl.ds(base, _BPW)], idx_v)
        pltpu.async_copy(table_hbm.at[idx_v], rows_v, sem).wait()
        pltpu.sync_copy(rows_v, out_hbm.at[pl.ds(base, _BPW)])

    return _sc_gather


def kernel(z, W):
    B, C, D, H, Wd = z.shape
    z3 = z.reshape(B, C, D * H * Wd)
    idx, minsum = _dist_argmin(z3, W)
    z_q_flat = _sc_gather_kernel()(W, idx)
    z_q = jnp.transpose(z_q_flat.reshape(B, D, H, Wd, C), (0, 4, 1, 2, 3))
    loss = (1.0 + CCOST) * (minsum[0, 0] / z.size)
    z_q_out = z + lax.stop_gradient(z_q - z)
    return (z_q_out, loss, idx.reshape(B, D, H, Wd))


# speedup vs baseline: 1.2106x; 1.0040x over previous
import functools

import jax
import jax.numpy as jnp
from jax import lax
from jax.experimental import pallas as pl
from jax.experimental.pallas import tpu as pltpu
from jax.experimental.pallas import tpu_sc as plsc

NUM_EMB = 8192
DIM = 256
CCOST = 0.25
TM = 1024
RC = 64

LANES = 128


def _dist_argmin_body(zt_ref, w_ref, idx_ref, minsum_ref, wsq_ref):
    b = pl.program_id(0)
    t = pl.program_id(1)
    first = (b == 0) & (t == 0)

    @pl.when(first)
    def _():
        w3 = w_ref[...].reshape(NUM_EMB // LANES, LANES, DIM)
        wsq_ref[...] = jnp.sum(w3 * w3, axis=2)

    z = jnp.transpose(zt_ref[0], (1, 0))
    zsq = jnp.sum(z * z, axis=1, keepdims=True)
    mm2 = lax.dot_general(z + z, w_ref[...], (((1,), (1,)), ((), ())),
                          preferred_element_type=jnp.float32)
    wsq = wsq_ref[...]

    lane = lax.broadcasted_iota(jnp.int32, (RC, LANES), 1)
    idx_parts, s = [], jnp.float32(0.0)
    for r in range(TM // RC):
        rows = slice(r * RC, (r + 1) * RC)
        zsq_r = zsq[rows]
        best = (zsq_r - mm2[rows, :LANES]) + wsq[0][None, :]
        bestj = jnp.zeros((RC, LANES), jnp.int32)
        for j in range(1, NUM_EMB // LANES):
            sj = (zsq_r - mm2[rows, j * LANES:(j + 1) * LANES]) + wsq[j][None, :]
            m = sj < best
            best = jnp.where(m, sj, best)
            bestj = jnp.where(m, j, bestj)
        absidx = bestj * LANES + lane
        rowmin = jnp.min(best, axis=1, keepdims=True)
        idx_parts.append(
            jnp.min(jnp.where(best == rowmin, absidx, NUM_EMB), axis=1))
        s = s + jnp.sum(rowmin)
    idx_ref[...] = jnp.concatenate(idx_parts)

    @pl.when(first)
    def _():
        minsum_ref[0, 0] = s

    @pl.when(jnp.logical_not(first))
    def _():
        minsum_ref[0, 0] = minsum_ref[0, 0] + s


def _dist_argmin(z3, w):
    nb = z3.shape[0]
    nt = z3.shape[2] // TM
    return pl.pallas_call(
        _dist_argmin_body,
        grid=(nb, nt),
        in_specs=[
            pl.BlockSpec((1, DIM, TM), lambda b, t: (b, 0, t)),
            pl.BlockSpec((NUM_EMB, DIM), lambda b, t: (0, 0)),
        ],
        out_specs=[
            pl.BlockSpec((TM,), lambda b, t: (b * nt + t,)),
            pl.BlockSpec((1, 1), lambda b, t: (0, 0), memory_space=pltpu.SMEM),
        ],
        out_shape=[
            jax.ShapeDtypeStruct((nb * nt * TM,), jnp.int32),
            jax.ShapeDtypeStruct((1, 1), jnp.float32),
        ],
        scratch_shapes=[pltpu.VMEM((NUM_EMB // LANES, LANES), jnp.float32)],
    )(z3, w)


_NC = 2
_NS = 16
_NW = _NC * _NS
_BPW = 8192 // _NW


@functools.cache
def _sc_gather_kernel():
    @functools.partial(
        pl.kernel,
        mesh=plsc.VectorSubcoreMesh(core_axis_name="c", subcore_axis_name="s"),
        out_type=jax.ShapeDtypeStruct((8192, DIM), jnp.float32),
        scratch_types=[
            pltpu.VMEM((_BPW,), jnp.int32),
            pltpu.VMEM((_BPW, DIM), jnp.float32),
            pltpu.SemaphoreType.DMA,
        ],
    )
    def _sc_gather(table_hbm, idx_hbm, out_hbm, idx_v, rows_v, sem):
        wid = lax.axis_index("s") * _NC + lax.axis_index("c")
        base = wid * _BPW
        pltpu.sync_copy(idx_hbm.at[pl.ds(base, _BPW)], idx_v)
        pltpu.async_copy(table_hbm.at[idx_v], rows_v, sem).wait()
        pltpu.sync_copy(rows_v, out_hbm.at[pl.ds(base, _BPW)])

    return _sc_gather


def kernel(z, W):
    B, C, D, H, Wd = z.shape
    z3 = z.reshape(B, C, D * H * Wd)
    idx, minsum = _dist_argmin(z3, W)
    z_q_flat = _sc_gather_kernel()(W, idx)
    z_q = jnp.transpose(z_q_flat.reshape(B, D, H, Wd, C), (0, 4, 1, 2, 3))
    loss = (1.0 + CCOST) * (minsum[0, 0] / z.size)
    z_q_out = z + lax.stop_gradient(z_q - z)
    return (z_q_out, loss, idx.reshape(B, D, H, Wd))
